# baseline (speedup 1.0000x reference)
import functools

import jax
import jax.numpy as jnp
from jax import lax
from jax.experimental import pallas as pl
from jax.experimental.pallas import tpu as pltpu
from jax.experimental.pallas import tpu_sc as plsc

V = 1000000
E = 64
H = 64
H2 = 64
C = 10
B = 1024
L = 200

F32 = jnp.float32
Bp = B // 2
Hp = 2 * H

_NC = 2
_NS = 16
_NW = _NC * _NS
_N = B * L
_BPW = _N // _NW
_CH = 640


def _sc_gather_body(table_hbm, idx_hbm, out_hbm, idx_v, rows_a, rows_b,
                    sem_a, sem_b):
    wid = lax.axis_index("s") * _NC + lax.axis_index("c")
    base = wid * _BPW
    pltpu.sync_copy(idx_hbm.at[pl.ds(base, _BPW)], idx_v)

    def step(j, carry):
        o = j * (2 * _CH)
        cp_a = pltpu.async_copy(table_hbm.at[idx_v.at[pl.ds(o, _CH)]], rows_a, sem_a)
        cp_b = pltpu.async_copy(
            table_hbm.at[idx_v.at[pl.ds(o + _CH, _CH)]], rows_b, sem_b)
        cp_a.wait()
        pltpu.sync_copy(rows_a, out_hbm.at[pl.ds(base + o, _CH)])
        cp_b.wait()
        pltpu.sync_copy(rows_b, out_hbm.at[pl.ds(base + o + _CH, _CH)])
        return carry

    lax.fori_loop(0, _BPW // (2 * _CH), step, 0)


@functools.cache
def _make_sc_gather():
    mesh = plsc.VectorSubcoreMesh(core_axis_name="c", subcore_axis_name="s")
    return pl.kernel(
        _sc_gather_body,
        mesh=mesh,
        out_type=jax.ShapeDtypeStruct((_N, E), jnp.bfloat16),
        scratch_types=[
            pltpu.VMEM((_BPW,), jnp.int32),
            pltpu.VMEM((_CH, E), jnp.bfloat16),
            pltpu.VMEM((_CH, E), jnp.bfloat16),
            pltpu.SemaphoreType.DMA,
            pltpu.SemaphoreType.DMA,
        ],
        compiler_params=pltpu.CompilerParams(use_tc_tiling_on_sc=False),
    )


def _pexp(Wt, G, D):
    n = Wt.shape[0]
    W4 = Wt.reshape(n, G, D)
    Z = jnp.zeros((2, n, G, 2, D), Wt.dtype)
    Z = Z.at[0, :, :, 0, :].set(W4)
    Z = Z.at[1, :, :, 1, :].set(W4)
    return Z.reshape(2 * n, G * 2 * D)


def _pbias(b, G, D):
    b4 = b.reshape(G, D)
    return jnp.stack([b4, b4], axis=1).reshape(1, G * 2 * D)


def _sig(x):
    return 0.5 * jnp.tanh(0.5 * x) + 0.5


def _cellp(g, c_prev):
    ig = _sig(g[:, :Hp])
    fg = _sig(g[:, Hp:2 * Hp])
    gg = jnp.tanh(g[:, 2 * Hp:3 * Hp])
    og = _sig(g[:, 3 * Hp:])
    c = fg * c_prev + ig * gg
    h = og * jnp.tanh(c)
    return h, c


def _dot(a, b):
    return jnp.dot(a, b, preferred_element_type=F32)


_TS = 8


def _l0_body(ef, eb, Wf, bf, Wr, br, hf_out, hb_out, hf_s, cf_s, hb_s, cb_s):
    t = pl.program_id(0)

    @pl.when(t == 0)
    def _init():
        hf_s[...] = jnp.zeros_like(hf_s)
        cf_s[...] = jnp.zeros_like(cf_s)
        hb_s[...] = jnp.zeros_like(hb_s)
        cb_s[...] = jnp.zeros_like(cb_s)

    for k in range(_TS):
        gf = _dot(jnp.concatenate([ef[k].astype(F32), hf_s[...]], axis=1),
                  Wf[...]) + bf[...]
        h, c = _cellp(gf, cf_s[...])
        hf_s[...] = h
        cf_s[...] = c
        hf_out[k] = h.astype(jnp.bfloat16)

        gb = _dot(jnp.concatenate([eb[_TS - 1 - k].astype(F32), hb_s[...]],
                                  axis=1), Wr[...]) + br[...]
        h, c = _cellp(gb, cb_s[...])
        hb_s[...] = h
        cb_s[...] = c
        hb_out[_TS - 1 - k] = h.astype(jnp.bfloat16)


def _l1_body(ff, bf_in, fb, bb, Wf, bf, Wr, br, hf_out, hb_out,
             hf_s, cf_s, hb_s, cb_s):
    t = pl.program_id(0)

    @pl.when(t == 0)
    def _init():
        hf_s[...] = jnp.zeros_like(hf_s)
        cf_s[...] = jnp.zeros_like(cf_s)
        hb_s[...] = jnp.zeros_like(hb_s)
        cb_s[...] = jnp.zeros_like(cb_s)

    for k in range(_TS):
        Xf = jnp.concatenate([ff[k].astype(F32), bf_in[k].astype(F32),
                              hf_s[...]], axis=1)
        h, c = _cellp(_dot(Xf, Wf[...]) + bf[...], cf_s[...])
        hf_s[...] = h
        cf_s[...] = c
        hf_out[k] = h.astype(jnp.bfloat16)

        Xb = jnp.concatenate([fb[_TS - 1 - k].astype(F32),
                              bb[_TS - 1 - k].astype(F32), hb_s[...]],
                             axis=1)
        h, c = _cellp(_dot(Xb, Wr[...]) + br[...], cb_s[...])
        hb_s[...] = h
        cb_s[...] = c
        hb_out[_TS - 1 - k] = h.astype(jnp.bfloat16)


def _seq_spec(fwd):
    if fwd:
        return pl.BlockSpec((_TS, Bp, Hp), lambda t: (t, 0, 0))
    return pl.BlockSpec((_TS, Bp, Hp), lambda t: (L // _TS - 1 - t, 0, 0))


def _w_spec(r, c):
    return pl.BlockSpec((r, c), lambda t: (0, 0))


def _run_l0(e2, Wf, bf, Wr, br):
    return pl.pallas_call(
        _l0_body,
        grid=(L // _TS,),
        in_specs=[
            _seq_spec(True), _seq_spec(False),
            _w_spec(2 * Hp, 4 * Hp), _w_spec(1, 4 * Hp),
            _w_spec(2 * Hp, 4 * Hp), _w_spec(1, 4 * Hp),
        ],
        out_specs=[_seq_spec(True), _seq_spec(False)],
        out_shape=[jax.ShapeDtypeStruct((L, Bp, Hp), jnp.bfloat16)] * 2,
        scratch_shapes=[pltpu.VMEM((Bp, Hp), F32)] * 4,
    )(e2, e2, Wf, bf, Wr, br)


def _run_l1(hf0, hb0, Wf, bf, Wr, br):
    return pl.pallas_call(
        _l1_body,
        grid=(L // _TS,),
        in_specs=[
            _seq_spec(True), _seq_spec(True),
            _seq_spec(False), _seq_spec(False),
            _w_spec(3 * Hp, 4 * Hp), _w_spec(1, 4 * Hp),
            _w_spec(3 * Hp, 4 * Hp), _w_spec(1, 4 * Hp),
        ],
        out_specs=[_seq_spec(True), _seq_spec(False)],
        out_shape=[jax.ShapeDtypeStruct((L, Bp, Hp), jnp.bfloat16)] * 2,
        scratch_shapes=[pltpu.VMEM((Bp, Hp), F32)] * 4,
    )(hf0, hb0, hf0, hb0, Wf, bf, Wr, br)


_BC2 = 64


def _att_body(hf, hb, wf2, wr2, Pm, w1a, w1b, b1, w2m, b2, out):
    f = hf[...].astype(F32)
    b_ = hb[...].astype(F32)
    spre = jnp.tanh(f) * wf2[...] + jnp.tanh(b_) * wr2[...]
    s = _dot(spre.reshape(L * _BC2, Hp), Pm[...]).reshape(L, _BC2, Hp)
    m = jnp.max(s, axis=0, keepdims=True)
    p = jnp.exp(s - m)
    a = p / jnp.sum(p, axis=0, keepdims=True)
    of = jnp.maximum(jnp.sum(f * a, axis=0), 0.0)
    ob = jnp.maximum(jnp.sum(b_ * a, axis=0), 0.0)
    h1 = _dot(of, w1a[...]) + _dot(ob, w1b[...]) + b1[...]
    out[...] = _dot(h1, w2m[...]) + b2[...]


def _run_att(hf1, hb1, wf2, wr2, Pm, w1a, w1b, b1, w2m, b2):
    chunk = pl.BlockSpec((L, _BC2, Hp), lambda i: (0, i, 0))
    return pl.pallas_call(
        _att_body,
        grid=(Bp // _BC2,),
        in_specs=[
            chunk, chunk,
            pl.BlockSpec((1, 1, Hp), lambda i: (0, 0, 0)),
            pl.BlockSpec((1, 1, Hp), lambda i: (0, 0, 0)),
            pl.BlockSpec((Hp, Hp), lambda i: (0, 0)),
            pl.BlockSpec((Hp, Hp), lambda i: (0, 0)),
            pl.BlockSpec((Hp, Hp), lambda i: (0, 0)),
            pl.BlockSpec((1, Hp), lambda i: (0, 0)),
            pl.BlockSpec((Hp, 2 * C), lambda i: (0, 0)),
            pl.BlockSpec((1, 2 * C), lambda i: (0, 0)),
        ],
        out_specs=pl.BlockSpec((_BC2, 2 * C), lambda i: (i, 0)),
        out_shape=jax.ShapeDtypeStruct((Bp, 2 * C), F32),
    )(hf1, hb1, wf2, wr2, Pm, w1a, w1b, b1, w2m, b2)


def _tc_forward(e2, Wih0f, Whh0f, bih0f, bhh0f, Wih0r, Whh0r, bih0r, bhh0r,
                Wih1f, Whh1f, bih1f, bhh1f, Wih1r, Whh1r, bih1r, bhh1r,
                w_att, fc1_w, fc1_b, fc_w, fc_b):
    W0f = jnp.concatenate(
        [_pexp(Wih0f.T, 4, H), _pexp(Whh0f.T, 4, H)], axis=0)
    W0r = jnp.concatenate(
        [_pexp(Wih0r.T, 4, H), _pexp(Whh0r.T, 4, H)], axis=0)
    b0f = _pbias(bih0f + bhh0f, 4, H)
    b0r = _pbias(bih0r + bhh0r, 4, H)
    hf0, hb0 = _run_l0(e2, W0f, b0f, W0r, b0r)

    W1ft = Wih1f.T
    W1rt = Wih1r.T
    W1f = jnp.concatenate(
        [_pexp(W1ft[:H], 4, H), _pexp(W1ft[H:], 4, H),
         _pexp(Whh1f.T, 4, H)], axis=0)
    W1r = jnp.concatenate(
        [_pexp(W1rt[:H], 4, H), _pexp(W1rt[H:], 4, H),
         _pexp(Whh1r.T, 4, H)], axis=0)
    b1f = _pbias(bih1f + bhh1f, 4, H)
    b1r = _pbias(bih1r + bhh1r, 4, H)
    hf1, hb1 = _run_l1(hf0, hb0, W1f, b1f, W1r, b1r)

    wf2 = jnp.concatenate([w_att[:H], w_att[:H]]).reshape(1, 1, Hp)
    wr2 = jnp.concatenate([w_att[H:], w_att[H:]]).reshape(1, 1, Hp)
    Pm = jnp.zeros((Hp, Hp), F32)
    Pm = Pm.at[:H, :H].set(1.0).at[H:, H:].set(1.0)
    w1t = fc1_w.T
    w1a = _pexp(w1t[:H], 1, H2)
    w1b = _pexp(w1t[H:], 1, H2)
    b1p = _pbias(fc1_b, 1, H2)
    w2m = _pexp(fc_w.T, 1, C)
    b2p = _pbias(fc_b, 1, C)
    out2 = _run_att(hf1, hb1, wf2, wr2, Pm, w1a, w1b, b1p, w2m, b2p)
    return out2.reshape(B, C)


def kernel(x, emb, Wih0f, Whh0f, bih0f, bhh0f, Wih0r, Whh0r, bih0r, bhh0r,
           Wih1f, Whh1f, bih1f, bhh1f, Wih1r, Whh1r, bih1r, bhh1r,
           w_att, fc1_w, fc1_b, fc_w, fc_b):
    idx = x.astype(jnp.int32).T.reshape(-1)
    e2 = _make_sc_gather()(emb.astype(jnp.bfloat16), idx).reshape(L, Bp, Hp)
    return _tc_forward(e2, Wih0f, Whh0f, bih0f, bhh0f, Wih0r, Whh0r,
                       bih0r, bhh0r, Wih1f, Whh1f, bih1f, bhh1f,
                       Wih1r, Whh1r, bih1r, bhh1r,
                       w_att, fc1_w, fc1_b, fc_w, fc_b)


# speedup vs baseline: 1.2096x; 1.2096x over previous
import functools

import jax
import jax.numpy as jnp
from jax import lax
from jax.experimental import pallas as pl
from jax.experimental.pallas import tpu as pltpu
from jax.experimental.pallas import tpu_sc as plsc

V = 1000000
E = 64
H = 64
H2 = 64
C = 10
B = 1024
L = 200

F32 = jnp.float32
Bp = B // 2
Hp = 2 * H

_NC = 2
_NS = 16
_NW = _NC * _NS
_N = B * L
_BPW = _N // _NW
_CH = 640


def _sc_gather_body(table_hbm, idx_hbm, out_hbm, idx_v, rows_a, rows_b,
                    sem_a, sem_b):
    wid = lax.axis_index("s") * _NC + lax.axis_index("c")
    base = wid * _BPW
    pltpu.sync_copy(idx_hbm.at[pl.ds(base, _BPW)], idx_v)

    def step(j, carry):
        o = j * (2 * _CH)
        cp_a = pltpu.async_copy(table_hbm.at[idx_v.at[pl.ds(o, _CH)]], rows_a, sem_a)
        cp_b = pltpu.async_copy(
            table_hbm.at[idx_v.at[pl.ds(o + _CH, _CH)]], rows_b, sem_b)
        cp_a.wait()
        pltpu.sync_copy(rows_a, out_hbm.at[pl.ds(base + o, _CH)])
        cp_b.wait()
        pltpu.sync_copy(rows_b, out_hbm.at[pl.ds(base + o + _CH, _CH)])
        return carry

    lax.fori_loop(0, _BPW // (2 * _CH), step, 0)


@functools.cache
def _make_sc_gather():
    mesh = plsc.VectorSubcoreMesh(core_axis_name="c", subcore_axis_name="s")
    return pl.kernel(
        _sc_gather_body,
        mesh=mesh,
        out_type=jax.ShapeDtypeStruct((_N, E), F32),
        scratch_types=[
            pltpu.VMEM((_BPW,), jnp.int32),
            pltpu.VMEM((_CH, E), F32),
            pltpu.VMEM((_CH, E), F32),
            pltpu.SemaphoreType.DMA,
            pltpu.SemaphoreType.DMA,
        ],
        compiler_params=pltpu.CompilerParams(use_tc_tiling_on_sc=False),
    )


def _pexp(Wt, G, D):
    n = Wt.shape[0]
    W4 = Wt.reshape(n, G, D)
    Z = jnp.zeros((2, n, G, 2, D), Wt.dtype)
    Z = Z.at[0, :, :, 0, :].set(W4)
    Z = Z.at[1, :, :, 1, :].set(W4)
    return Z.reshape(2 * n, G * 2 * D)


def _pbias(b, G, D):
    b4 = b.reshape(G, D)
    return jnp.stack([b4, b4], axis=1).reshape(1, G * 2 * D)


def _sig(x):
    return 0.5 * jnp.tanh(0.5 * x) + 0.5


def _cellp(g, c_prev):
    ig = _sig(g[:, :Hp])
    fg = _sig(g[:, Hp:2 * Hp])
    gg = jnp.tanh(g[:, 2 * Hp:3 * Hp])
    og = _sig(g[:, 3 * Hp:])
    c = fg * c_prev + ig * gg
    h = og * jnp.tanh(c)
    return h, c


def _dot(a, b):
    return jnp.dot(a, b, preferred_element_type=F32)


_TS = 8


def _l0_body(ef, eb, Wf, bf, Wr, br, hf_out, hb_out, hf_s, cf_s, hb_s, cb_s):
    t = pl.program_id(0)

    @pl.when(t == 0)
    def _init():
        hf_s[...] = jnp.zeros_like(hf_s)
        cf_s[...] = jnp.zeros_like(cf_s)
        hb_s[...] = jnp.zeros_like(hb_s)
        cb_s[...] = jnp.zeros_like(cb_s)

    for k in range(_TS):
        gf = (_dot(ef[k], Wf[:Hp]) + _dot(hf_s[...], Wf[Hp:]) + bf[...])
        h, c = _cellp(gf, cf_s[...])
        hf_s[...] = h
        cf_s[...] = c
        hf_out[k] = h.astype(jnp.bfloat16)

        gb = (_dot(eb[_TS - 1 - k], Wr[:Hp]) + _dot(hb_s[...], Wr[Hp:])
              + br[...])
        h, c = _cellp(gb, cb_s[...])
        hb_s[...] = h
        cb_s[...] = c
        hb_out[_TS - 1 - k] = h.astype(jnp.bfloat16)


def _l1_body(ff, bf_in, fb, bb, Wf, bf, Wr, br, hf_out, hb_out,
             hf_s, cf_s, hb_s, cb_s):
    t = pl.program_id(0)

    @pl.when(t == 0)
    def _init():
        hf_s[...] = jnp.zeros_like(hf_s)
        cf_s[...] = jnp.zeros_like(cf_s)
        hb_s[...] = jnp.zeros_like(hb_s)
        cb_s[...] = jnp.zeros_like(cb_s)

    for k in range(_TS):
        gf = (_dot(ff[k].astype(F32), Wf[:Hp])
              + _dot(bf_in[k].astype(F32), Wf[Hp:2 * Hp])
              + _dot(hf_s[...], Wf[2 * Hp:]) + bf[...])
        h, c = _cellp(gf, cf_s[...])
        hf_s[...] = h
        cf_s[...] = c
        hf_out[k] = h.astype(jnp.bfloat16)

        gb = (_dot(fb[_TS - 1 - k].astype(F32), Wr[:Hp])
              + _dot(bb[_TS - 1 - k].astype(F32), Wr[Hp:2 * Hp])
              + _dot(hb_s[...], Wr[2 * Hp:]) + br[...])
        h, c = _cellp(gb, cb_s[...])
        hb_s[...] = h
        cb_s[...] = c
        hb_out[_TS - 1 - k] = h.astype(jnp.bfloat16)


def _seq_spec(fwd):
    if fwd:
        return pl.BlockSpec((_TS, Bp, Hp), lambda t: (t, 0, 0))
    return pl.BlockSpec((_TS, Bp, Hp), lambda t: (L // _TS - 1 - t, 0, 0))


def _w_spec(r, c):
    return pl.BlockSpec((r, c), lambda t: (0, 0))


def _run_l0(e2, Wf, bf, Wr, br):
    return pl.pallas_call(
        _l0_body,
        grid=(L // _TS,),
        in_specs=[
            _seq_spec(True), _seq_spec(False),
            _w_spec(2 * Hp, 4 * Hp), _w_spec(1, 4 * Hp),
            _w_spec(2 * Hp, 4 * Hp), _w_spec(1, 4 * Hp),
        ],
        out_specs=[_seq_spec(True), _seq_spec(False)],
        out_shape=[jax.ShapeDtypeStruct((L, Bp, Hp), jnp.bfloat16)] * 2,
        scratch_shapes=[pltpu.VMEM((Bp, Hp), F32)] * 4,
    )(e2, e2, Wf, bf, Wr, br)


def _run_l1(hf0, hb0, Wf, bf, Wr, br):
    return pl.pallas_call(
        _l1_body,
        grid=(L // _TS,),
        in_specs=[
            _seq_spec(True), _seq_spec(True),
            _seq_spec(False), _seq_spec(False),
            _w_spec(3 * Hp, 4 * Hp), _w_spec(1, 4 * Hp),
            _w_spec(3 * Hp, 4 * Hp), _w_spec(1, 4 * Hp),
        ],
        out_specs=[_seq_spec(True), _seq_spec(False)],
        out_shape=[jax.ShapeDtypeStruct((L, Bp, Hp), jnp.bfloat16)] * 2,
        scratch_shapes=[pltpu.VMEM((Bp, Hp), F32)] * 4,
    )(hf0, hb0, hf0, hb0, Wf, bf, Wr, br)


_BC2 = 64


def _att_body(hf, hb, wf2, wr2, Pm, w1a, w1b, b1, w2m, b2, out):
    f = hf[...].astype(F32)
    b_ = hb[...].astype(F32)
    spre = jnp.tanh(f) * wf2[...] + jnp.tanh(b_) * wr2[...]
    s = _dot(spre.reshape(L * _BC2, Hp), Pm[...]).reshape(L, _BC2, Hp)
    m = jnp.max(s, axis=0, keepdims=True)
    p = jnp.exp(s - m)
    a = p / jnp.sum(p, axis=0, keepdims=True)
    of = jnp.maximum(jnp.sum(f * a, axis=0), 0.0)
    ob = jnp.maximum(jnp.sum(b_ * a, axis=0), 0.0)
    h1 = _dot(of, w1a[...]) + _dot(ob, w1b[...]) + b1[...]
    out[...] = _dot(h1, w2m[...]) + b2[...]


def _run_att(hf1, hb1, wf2, wr2, Pm, w1a, w1b, b1, w2m, b2):
    chunk = pl.BlockSpec((L, _BC2, Hp), lambda i: (0, i, 0))
    return pl.pallas_call(
        _att_body,
        grid=(Bp // _BC2,),
        in_specs=[
            chunk, chunk,
            pl.BlockSpec((1, 1, Hp), lambda i: (0, 0, 0)),
            pl.BlockSpec((1, 1, Hp), lambda i: (0, 0, 0)),
            pl.BlockSpec((Hp, Hp), lambda i: (0, 0)),
            pl.BlockSpec((Hp, Hp), lambda i: (0, 0)),
            pl.BlockSpec((Hp, Hp), lambda i: (0, 0)),
            pl.BlockSpec((1, Hp), lambda i: (0, 0)),
            pl.BlockSpec((Hp, 2 * C), lambda i: (0, 0)),
            pl.BlockSpec((1, 2 * C), lambda i: (0, 0)),
        ],
        out_specs=pl.BlockSpec((_BC2, 2 * C), lambda i: (i, 0)),
        out_shape=jax.ShapeDtypeStruct((Bp, 2 * C), F32),
    )(hf1, hb1, wf2, wr2, Pm, w1a, w1b, b1, w2m, b2)


def _tc_forward(e2, Wih0f, Whh0f, bih0f, bhh0f, Wih0r, Whh0r, bih0r, bhh0r,
                Wih1f, Whh1f, bih1f, bhh1f, Wih1r, Whh1r, bih1r, bhh1r,
                w_att, fc1_w, fc1_b, fc_w, fc_b):
    W0f = jnp.concatenate(
        [_pexp(Wih0f.T, 4, H), _pexp(Whh0f.T, 4, H)], axis=0)
    W0r = jnp.concatenate(
        [_pexp(Wih0r.T, 4, H), _pexp(Whh0r.T, 4, H)], axis=0)
    b0f = _pbias(bih0f + bhh0f, 4, H)
    b0r = _pbias(bih0r + bhh0r, 4, H)
    hf0, hb0 = _run_l0(e2, W0f, b0f, W0r, b0r)

    W1ft = Wih1f.T
    W1rt = Wih1r.T
    W1f = jnp.concatenate(
        [_pexp(W1ft[:H], 4, H), _pexp(W1ft[H:], 4, H),
         _pexp(Whh1f.T, 4, H)], axis=0)
    W1r = jnp.concatenate(
        [_pexp(W1rt[:H], 4, H), _pexp(W1rt[H:], 4, H),
         _pexp(Whh1r.T, 4, H)], axis=0)
    b1f = _pbias(bih1f + bhh1f, 4, H)
    b1r = _pbias(bih1r + bhh1r, 4, H)
    hf1, hb1 = _run_l1(hf0, hb0, W1f, b1f, W1r, b1r)

    wf2 = jnp.concatenate([w_att[:H], w_att[:H]]).reshape(1, 1, Hp)
    wr2 = jnp.concatenate([w_att[H:], w_att[H:]]).reshape(1, 1, Hp)
    Pm = jnp.zeros((Hp, Hp), F32)
    Pm = Pm.at[:H, :H].set(1.0).at[H:, H:].set(1.0)
    w1t = fc1_w.T
    w1a = _pexp(w1t[:H], 1, H2)
    w1b = _pexp(w1t[H:], 1, H2)
    b1p = _pbias(fc1_b, 1, H2)
    w2m = _pexp(fc_w.T, 1, C)
    b2p = _pbias(fc_b, 1, C)
    out2 = _run_att(hf1, hb1, wf2, wr2, Pm, w1a, w1b, b1p, w2m, b2p)
    return out2.reshape(B, C)


def kernel(x, emb, Wih0f, Whh0f, bih0f, bhh0f, Wih0r, Whh0r, bih0r, bhh0r,
           Wih1f, Whh1f, bih1f, bhh1f, Wih1r, Whh1r, bih1r, bhh1r,
           w_att, fc1_w, fc1_b, fc_w, fc_b):
    idx = x.astype(jnp.int32).T.reshape(-1)
    e2 = _make_sc_gather()(emb, idx).reshape(L, Bp, Hp)
    return _tc_forward(e2, Wih0f, Whh0f, bih0f, bhh0f, Wih0r, Whh0r,
                       bih0r, bhh0r, Wih1f, Whh1f, bih1f, bhh1f,
                       Wih1r, Whh1r, bih1r, bhh1r,
                       w_att, fc1_w, fc1_b, fc_w, fc_b)


# speedup vs baseline: 1.3206x; 1.0918x over previous
import functools

import jax
import jax.numpy as jnp
from jax import lax
from jax.experimental import pallas as pl
from jax.experimental.pallas import tpu as pltpu
from jax.experimental.pallas import tpu_sc as plsc

V = 1000000
E = 64
H = 64
H2 = 64
C = 10
B = 1024
L = 200

F32 = jnp.float32
Bp = B // 2
Hp = 2 * H

_NC = 2
_NS = 16
_NW = _NC * _NS
_N = B * L
_BPW = _N // _NW
_CH = 640


def _sc_gather_body(table_hbm, idx_hbm, out_hbm, idx_v, rows_a, rows_b,
                    sem_a, sem_b):
    wid = lax.axis_index("s") * _NC + lax.axis_index("c")
    base = wid * _BPW
    pltpu.sync_copy(idx_hbm.at[pl.ds(base, _BPW)], idx_v)

    def step(j, carry):
        o = j * (2 * _CH)
        cp_a = pltpu.async_copy(table_hbm.at[idx_v.at[pl.ds(o, _CH)]], rows_a, sem_a)
        cp_b = pltpu.async_copy(
            table_hbm.at[idx_v.at[pl.ds(o + _CH, _CH)]], rows_b, sem_b)
        cp_a.wait()
        pltpu.sync_copy(rows_a, out_hbm.at[pl.ds(base + o, _CH)])
        cp_b.wait()
        pltpu.sync_copy(rows_b, out_hbm.at[pl.ds(base + o + _CH, _CH)])
        return carry

    lax.fori_loop(0, _BPW // (2 * _CH), step, 0)


@functools.cache
def _make_sc_gather():
    mesh = plsc.VectorSubcoreMesh(core_axis_name="c", subcore_axis_name="s")
    return pl.kernel(
        _sc_gather_body,
        mesh=mesh,
        out_type=jax.ShapeDtypeStruct((_N, E), F32),
        scratch_types=[
            pltpu.VMEM((_BPW,), jnp.int32),
            pltpu.VMEM((_CH, E), F32),
            pltpu.VMEM((_CH, E), F32),
            pltpu.SemaphoreType.DMA,
            pltpu.SemaphoreType.DMA,
        ],
        compiler_params=pltpu.CompilerParams(use_tc_tiling_on_sc=False),
    )


def _pexp(Wt, G, D):
    n = Wt.shape[0]
    W4 = Wt.reshape(n, G, D)
    Z = jnp.zeros((2, n, G, 2, D), Wt.dtype)
    Z = Z.at[0, :, :, 0, :].set(W4)
    Z = Z.at[1, :, :, 1, :].set(W4)
    return Z.reshape(2 * n, G * 2 * D)


def _pbias(b, G, D):
    b4 = b.reshape(G, D)
    return jnp.stack([b4, b4], axis=1).reshape(1, G * 2 * D)


def _sig(x):
    return 0.5 * jnp.tanh(0.5 * x) + 0.5


def _cellp(g, c_prev):
    ig = _sig(g[:, :Hp])
    fg = _sig(g[:, Hp:2 * Hp])
    gg = jnp.tanh(g[:, 2 * Hp:3 * Hp])
    og = _sig(g[:, 3 * Hp:])
    c = fg * c_prev + ig * gg
    h = og * jnp.tanh(c)
    return h, c


def _dot(a, b):
    return jnp.dot(a, b, preferred_element_type=F32)


_TS = 8


def _l0_body(ef, eb, Wf, bf, Wr, br, hf_out, hb_out, hf_s, cf_s, hb_s, cb_s):
    t = pl.program_id(0)

    @pl.when(t == 0)
    def _init():
        hf_s[...] = jnp.zeros_like(hf_s)
        cf_s[...] = jnp.zeros_like(cf_s)
        hb_s[...] = jnp.zeros_like(hb_s)
        cb_s[...] = jnp.zeros_like(cb_s)

    for k in range(_TS):
        gf = _dot(jnp.concatenate([ef[k], hf_s[...]], axis=1), Wf[...]) + bf[...]
        h, c = _cellp(gf, cf_s[...])
        hf_s[...] = h
        cf_s[...] = c
        hf_out[k] = h.astype(jnp.bfloat16)

        gb = _dot(jnp.concatenate([eb[_TS - 1 - k], hb_s[...]], axis=1),
                  Wr[...]) + br[...]
        h, c = _cellp(gb, cb_s[...])
        hb_s[...] = h
        cb_s[...] = c
        hb_out[_TS - 1 - k] = h.astype(jnp.bfloat16)


def _l1_body(ff, bf_in, fb, bb, Wf, bf, Wr, br, hf_out, hb_out,
             hf_s, cf_s, hb_s, cb_s):
    t = pl.program_id(0)

    @pl.when(t == 0)
    def _init():
        hf_s[...] = jnp.zeros_like(hf_s)
        cf_s[...] = jnp.zeros_like(cf_s)
        hb_s[...] = jnp.zeros_like(hb_s)
        cb_s[...] = jnp.zeros_like(cb_s)

    for k in range(_TS):
        Xf = jnp.concatenate([ff[k].astype(F32), bf_in[k].astype(F32),
                              hf_s[...]], axis=1)
        h, c = _cellp(_dot(Xf, Wf[...]) + bf[...], cf_s[...])
        hf_s[...] = h
        cf_s[...] = c
        hf_out[k] = h.astype(jnp.bfloat16)

        Xb = jnp.concatenate([fb[_TS - 1 - k].astype(F32),
                              bb[_TS - 1 - k].astype(F32), hb_s[...]],
                             axis=1)
        h, c = _cellp(_dot(Xb, Wr[...]) + br[...], cb_s[...])
        hb_s[...] = h
        cb_s[...] = c
        hb_out[_TS - 1 - k] = h.astype(jnp.bfloat16)


def _seq_spec(fwd):
    if fwd:
        return pl.BlockSpec((_TS, Bp, Hp), lambda t: (t, 0, 0))
    return pl.BlockSpec((_TS, Bp, Hp), lambda t: (L // _TS - 1 - t, 0, 0))


def _w_spec(r, c):
    return pl.BlockSpec((r, c), lambda t: (0, 0))


def _run_l0(e2, Wf, bf, Wr, br):
    return pl.pallas_call(
        _l0_body,
        grid=(L // _TS,),
        in_specs=[
            _seq_spec(True), _seq_spec(False),
            _w_spec(2 * Hp, 4 * Hp), _w_spec(1, 4 * Hp),
            _w_spec(2 * Hp, 4 * Hp), _w_spec(1, 4 * Hp),
        ],
        out_specs=[_seq_spec(True), _seq_spec(False)],
        out_shape=[jax.ShapeDtypeStruct((L, Bp, Hp), jnp.bfloat16)] * 2,
        scratch_shapes=[pltpu.VMEM((Bp, Hp), F32)] * 4,
    )(e2, e2, Wf, bf, Wr, br)


def _run_l1(hf0, hb0, Wf, bf, Wr, br):
    return pl.pallas_call(
        _l1_body,
        grid=(L // _TS,),
        in_specs=[
            _seq_spec(True), _seq_spec(True),
            _seq_spec(False), _seq_spec(False),
            _w_spec(3 * Hp, 4 * Hp), _w_spec(1, 4 * Hp),
            _w_spec(3 * Hp, 4 * Hp), _w_spec(1, 4 * Hp),
        ],
        out_specs=[_seq_spec(True), _seq_spec(False)],
        out_shape=[jax.ShapeDtypeStruct((L, Bp, Hp), jnp.bfloat16)] * 2,
        scratch_shapes=[pltpu.VMEM((Bp, Hp), F32)] * 4,
    )(hf0, hb0, hf0, hb0, Wf, bf, Wr, br)


_BC2 = 64


def _att_body(hf, hb, wf2, wr2, Pm, w1a, w1b, b1, w2m, b2, out):
    f = hf[...].astype(F32)
    b_ = hb[...].astype(F32)
    spre = jnp.tanh(f) * wf2[...] + jnp.tanh(b_) * wr2[...]
    s = _dot(spre.reshape(L * _BC2, Hp), Pm[...]).reshape(L, _BC2, Hp)
    m = jnp.max(s, axis=0, keepdims=True)
    p = jnp.exp(s - m)
    a = p / jnp.sum(p, axis=0, keepdims=True)
    of = jnp.maximum(jnp.sum(f * a, axis=0), 0.0)
    ob = jnp.maximum(jnp.sum(b_ * a, axis=0), 0.0)
    h1 = _dot(of, w1a[...]) + _dot(ob, w1b[...]) + b1[...]
    out[...] = _dot(h1, w2m[...]) + b2[...]


def _run_att(hf1, hb1, wf2, wr2, Pm, w1a, w1b, b1, w2m, b2):
    chunk = pl.BlockSpec((L, _BC2, Hp), lambda i: (0, i, 0))
    return pl.pallas_call(
        _att_body,
        grid=(Bp // _BC2,),
        in_specs=[
            chunk, chunk,
            pl.BlockSpec((1, 1, Hp), lambda i: (0, 0, 0)),
            pl.BlockSpec((1, 1, Hp), lambda i: (0, 0, 0)),
            pl.BlockSpec((Hp, Hp), lambda i: (0, 0)),
            pl.BlockSpec((Hp, Hp), lambda i: (0, 0)),
            pl.BlockSpec((Hp, Hp), lambda i: (0, 0)),
            pl.BlockSpec((1, Hp), lambda i: (0, 0)),
            pl.BlockSpec((Hp, 2 * C), lambda i: (0, 0)),
            pl.BlockSpec((1, 2 * C), lambda i: (0, 0)),
        ],
        out_specs=pl.BlockSpec((_BC2, 2 * C), lambda i: (i, 0)),
        out_shape=jax.ShapeDtypeStruct((Bp, 2 * C), F32),
    )(hf1, hb1, wf2, wr2, Pm, w1a, w1b, b1, w2m, b2)


def _tc_forward(e2, Wih0f, Whh0f, bih0f, bhh0f, Wih0r, Whh0r, bih0r, bhh0r,
                Wih1f, Whh1f, bih1f, bhh1f, Wih1r, Whh1r, bih1r, bhh1r,
                w_att, fc1_w, fc1_b, fc_w, fc_b):
    W0f = jnp.concatenate(
        [_pexp(Wih0f.T, 4, H), _pexp(Whh0f.T, 4, H)], axis=0)
    W0r = jnp.concatenate(
        [_pexp(Wih0r.T, 4, H), _pexp(Whh0r.T, 4, H)], axis=0)
    b0f = _pbias(bih0f + bhh0f, 4, H)
    b0r = _pbias(bih0r + bhh0r, 4, H)
    hf0, hb0 = _run_l0(e2, W0f, b0f, W0r, b0r)

    W1ft = Wih1f.T
    W1rt = Wih1r.T
    W1f = jnp.concatenate(
        [_pexp(W1ft[:H], 4, H), _pexp(W1ft[H:], 4, H),
         _pexp(Whh1f.T, 4, H)], axis=0)
    W1r = jnp.concatenate(
        [_pexp(W1rt[:H], 4, H), _pexp(W1rt[H:], 4, H),
         _pexp(Whh1r.T, 4, H)], axis=0)
    b1f = _pbias(bih1f + bhh1f, 4, H)
    b1r = _pbias(bih1r + bhh1r, 4, H)
    hf1, hb1 = _run_l1(hf0, hb0, W1f, b1f, W1r, b1r)

    wf2 = jnp.concatenate([w_att[:H], w_att[:H]]).reshape(1, 1, Hp)
    wr2 = jnp.concatenate([w_att[H:], w_att[H:]]).reshape(1, 1, Hp)
    Pm = jnp.zeros((Hp, Hp), F32)
    Pm = Pm.at[:H, :H].set(1.0).at[H:, H:].set(1.0)
    w1t = fc1_w.T
    w1a = _pexp(w1t[:H], 1, H2)
    w1b = _pexp(w1t[H:], 1, H2)
    b1p = _pbias(fc1_b, 1, H2)
    w2m = _pexp(fc_w.T, 1, C)
    b2p = _pbias(fc_b, 1, C)
    out2 = _run_att(hf1, hb1, wf2, wr2, Pm, w1a, w1b, b1p, w2m, b2p)
    return out2.reshape(B, C)


def kernel(x, emb, Wih0f, Whh0f, bih0f, bhh0f, Wih0r, Whh0r, bih0r, bhh0r,
           Wih1f, Whh1f, bih1f, bhh1f, Wih1r, Whh1r, bih1r, bhh1r,
           w_att, fc1_w, fc1_b, fc_w, fc_b):
    idx = x.astype(jnp.int32).T.reshape(-1)
    e2 = _make_sc_gather()(emb, idx).reshape(L, Bp, Hp)
    return _tc_forward(e2, Wih0f, Whh0f, bih0f, bhh0f, Wih0r, Whh0r,
                       bih0r, bhh0r, Wih1f, Whh1f, bih1f, bhh1f,
                       Wih1r, Whh1r, bih1r, bhh1r,
                       w_att, fc1_w, fc1_b, fc_w, fc_b)


# speedup vs baseline: 1.3224x; 1.0013x over previous
import functools

import jax
import jax.numpy as jnp
from jax import lax
from jax.experimental import pallas as pl
from jax.experimental.pallas import tpu as pltpu
from jax.experimental.pallas import tpu_sc as plsc

V = 1000000
E = 64
H = 64
H2 = 64
C = 10
B = 1024
L = 200

F32 = jnp.float32
Bp = B // 2
Hp = 2 * H

_NC = 2
_NS = 16
_NW = _NC * _NS
_N = B * L
_BPW = _N // _NW
_CH = 640


def _sc_gather_body(table_hbm, idx_hbm, out_hbm, idx_v, rows_a, rows_b,
                    sem_a, sem_b):
    wid = lax.axis_index("s") * _NC + lax.axis_index("c")
    base = wid * _BPW
    pltpu.sync_copy(idx_hbm.at[pl.ds(base, _BPW)], idx_v)

    def step(j, carry):
        o = j * (2 * _CH)
        cp_a = pltpu.async_copy(table_hbm.at[idx_v.at[pl.ds(o, _CH)]], rows_a, sem_a)
        cp_b = pltpu.async_copy(
            table_hbm.at[idx_v.at[pl.ds(o + _CH, _CH)]], rows_b, sem_b)
        cp_a.wait()
        pltpu.sync_copy(rows_a, out_hbm.at[pl.ds(base + o, _CH)])
        cp_b.wait()
        pltpu.sync_copy(rows_b, out_hbm.at[pl.ds(base + o + _CH, _CH)])
        return carry

    lax.fori_loop(0, _BPW // (2 * _CH), step, 0)


@functools.cache
def _make_sc_gather():
    mesh = plsc.VectorSubcoreMesh(core_axis_name="c", subcore_axis_name="s")
    return pl.kernel(
        _sc_gather_body,
        mesh=mesh,
        out_type=jax.ShapeDtypeStruct((_N, E), F32),
        scratch_types=[
            pltpu.VMEM((_BPW,), jnp.int32),
            pltpu.VMEM((_CH, E), F32),
            pltpu.VMEM((_CH, E), F32),
            pltpu.SemaphoreType.DMA,
            pltpu.SemaphoreType.DMA,
        ],
        compiler_params=pltpu.CompilerParams(use_tc_tiling_on_sc=False),
    )


def _pexp(Wt, G, D):
    n = Wt.shape[0]
    W4 = Wt.reshape(n, G, D)
    Z = jnp.zeros((2, n, G, 2, D), Wt.dtype)
    Z = Z.at[0, :, :, 0, :].set(W4)
    Z = Z.at[1, :, :, 1, :].set(W4)
    return Z.reshape(2 * n, G * 2 * D)


def _pbias(b, G, D):
    b4 = b.reshape(G, D)
    return jnp.stack([b4, b4], axis=1).reshape(1, G * 2 * D)


def _sig(x):
    return 0.5 * jnp.tanh(0.5 * x) + 0.5


def _cellp(g, c_prev):
    ig = _sig(g[:, :Hp])
    fg = _sig(g[:, Hp:2 * Hp])
    gg = jnp.tanh(g[:, 2 * Hp:3 * Hp])
    og = _sig(g[:, 3 * Hp:])
    c = fg * c_prev + ig * gg
    h = og * jnp.tanh(c)
    return h, c


def _dot(a, b):
    return jnp.dot(a, b, preferred_element_type=F32)


_TS = 8


def _l0_body(ef, eb, Wf, bf, Wr, br, hf_out, hb_out, hf_s, cf_s, hb_s, cb_s):
    t = pl.program_id(0)

    @pl.when(t == 0)
    def _init():
        hf_s[...] = jnp.zeros_like(hf_s)
        cf_s[...] = jnp.zeros_like(cf_s)
        hb_s[...] = jnp.zeros_like(hb_s)
        cb_s[...] = jnp.zeros_like(cb_s)

    for k in range(_TS):
        gf = _dot(jnp.concatenate([ef[k], hf_s[...]], axis=1), Wf[...]) + bf[...]
        h, c = _cellp(gf, cf_s[...])
        hf_s[...] = h
        cf_s[...] = c
        hf_out[k] = h.astype(jnp.bfloat16)

        gb = _dot(jnp.concatenate([eb[_TS - 1 - k], hb_s[...]], axis=1),
                  Wr[...]) + br[...]
        h, c = _cellp(gb, cb_s[...])
        hb_s[...] = h
        cb_s[...] = c
        hb_out[_TS - 1 - k] = h.astype(jnp.bfloat16)


def _l1_body(ff, bf_in, fb, bb, Wf, bf, Wr, br, hf_out, hb_out,
             hf_s, cf_s, hb_s, cb_s):
    t = pl.program_id(0)

    @pl.when(t == 0)
    def _init():
        hf_s[...] = jnp.zeros_like(hf_s)
        cf_s[...] = jnp.zeros_like(cf_s)
        hb_s[...] = jnp.zeros_like(hb_s)
        cb_s[...] = jnp.zeros_like(cb_s)

    for k in range(_TS):
        Xf = jnp.concatenate([ff[k].astype(F32), bf_in[k].astype(F32),
                              hf_s[...]], axis=1)
        h, c = _cellp(_dot(Xf, Wf[...]) + bf[...], cf_s[...])
        hf_s[...] = h
        cf_s[...] = c
        hf_out[k] = h.astype(jnp.bfloat16)

        Xb = jnp.concatenate([fb[_TS - 1 - k].astype(F32),
                              bb[_TS - 1 - k].astype(F32), hb_s[...]],
                             axis=1)
        h, c = _cellp(_dot(Xb, Wr[...]) + br[...], cb_s[...])
        hb_s[...] = h
        cb_s[...] = c
        hb_out[_TS - 1 - k] = h.astype(jnp.bfloat16)


def _seq_spec(fwd):
    if fwd:
        return pl.BlockSpec((_TS, Bp, Hp), lambda t: (t, 0, 0))
    return pl.BlockSpec((_TS, Bp, Hp), lambda t: (L // _TS - 1 - t, 0, 0))


def _w_spec(r, c):
    return pl.BlockSpec((r, c), lambda t: (0, 0))


def _run_l0(e2, Wf, bf, Wr, br):
    return pl.pallas_call(
        _l0_body,
        grid=(L // _TS,),
        in_specs=[
            _seq_spec(True), _seq_spec(False),
            _w_spec(2 * Hp, 4 * Hp), _w_spec(1, 4 * Hp),
            _w_spec(2 * Hp, 4 * Hp), _w_spec(1, 4 * Hp),
        ],
        out_specs=[_seq_spec(True), _seq_spec(False)],
        out_shape=[jax.ShapeDtypeStruct((L, Bp, Hp), jnp.bfloat16)] * 2,
        scratch_shapes=[pltpu.VMEM((Bp, Hp), F32)] * 4,
    )(e2, e2, Wf, bf, Wr, br)


def _run_l1(hf0, hb0, Wf, bf, Wr, br):
    return pl.pallas_call(
        _l1_body,
        grid=(L // _TS,),
        in_specs=[
            _seq_spec(True), _seq_spec(True),
            _seq_spec(False), _seq_spec(False),
            _w_spec(3 * Hp, 4 * Hp), _w_spec(1, 4 * Hp),
            _w_spec(3 * Hp, 4 * Hp), _w_spec(1, 4 * Hp),
        ],
        out_specs=[_seq_spec(True), _seq_spec(False)],
        out_shape=[jax.ShapeDtypeStruct((L, Bp, Hp), jnp.bfloat16)] * 2,
        scratch_shapes=[pltpu.VMEM((Bp, Hp), F32)] * 4,
    )(hf0, hb0, hf0, hb0, Wf, bf, Wr, br)


_BC2 = 128


def _att_body(hf, hb, wf2, wr2, Pm, w1a, w1b, b1, w2m, b2, out):
    f = hf[...].astype(F32)
    b_ = hb[...].astype(F32)
    spre = jnp.tanh(f) * wf2[...] + jnp.tanh(b_) * wr2[...]
    s = _dot(spre.reshape(L * _BC2, Hp), Pm[...]).reshape(L, _BC2, Hp)
    m = jnp.max(s, axis=0, keepdims=True)
    p = jnp.exp(s - m)
    a = p / jnp.sum(p, axis=0, keepdims=True)
    of = jnp.maximum(jnp.sum(f * a, axis=0), 0.0)
    ob = jnp.maximum(jnp.sum(b_ * a, axis=0), 0.0)
    h1 = _dot(of, w1a[...]) + _dot(ob, w1b[...]) + b1[...]
    out[...] = _dot(h1, w2m[...]) + b2[...]


def _run_att(hf1, hb1, wf2, wr2, Pm, w1a, w1b, b1, w2m, b2):
    chunk = pl.BlockSpec((L, _BC2, Hp), lambda i: (0, i, 0))
    return pl.pallas_call(
        _att_body,
        grid=(Bp // _BC2,),
        in_specs=[
            chunk, chunk,
            pl.BlockSpec((1, 1, Hp), lambda i: (0, 0, 0)),
            pl.BlockSpec((1, 1, Hp), lambda i: (0, 0, 0)),
            pl.BlockSpec((Hp, Hp), lambda i: (0, 0)),
            pl.BlockSpec((Hp, Hp), lambda i: (0, 0)),
            pl.BlockSpec((Hp, Hp), lambda i: (0, 0)),
            pl.BlockSpec((1, Hp), lambda i: (0, 0)),
            pl.BlockSpec((Hp, 2 * C), lambda i: (0, 0)),
            pl.BlockSpec((1, 2 * C), lambda i: (0, 0)),
        ],
        out_specs=pl.BlockSpec((_BC2, 2 * C), lambda i: (i, 0)),
        out_shape=jax.ShapeDtypeStruct((Bp, 2 * C), F32),
    )(hf1, hb1, wf2, wr2, Pm, w1a, w1b, b1, w2m, b2)


def _tc_forward(e2, Wih0f, Whh0f, bih0f, bhh0f, Wih0r, Whh0r, bih0r, bhh0r,
                Wih1f, Whh1f, bih1f, bhh1f, Wih1r, Whh1r, bih1r, bhh1r,
                w_att, fc1_w, fc1_b, fc_w, fc_b):
    W0f = jnp.concatenate(
        [_pexp(Wih0f.T, 4, H), _pexp(Whh0f.T, 4, H)], axis=0)
    W0r = jnp.concatenate(
        [_pexp(Wih0r.T, 4, H), _pexp(Whh0r.T, 4, H)], axis=0)
    b0f = _pbias(bih0f + bhh0f, 4, H)
    b0r = _pbias(bih0r + bhh0r, 4, H)
    hf0, hb0 = _run_l0(e2, W0f, b0f, W0r, b0r)

    W1ft = Wih1f.T
    W1rt = Wih1r.T
    W1f = jnp.concatenate(
        [_pexp(W1ft[:H], 4, H), _pexp(W1ft[H:], 4, H),
         _pexp(Whh1f.T, 4, H)], axis=0)
    W1r = jnp.concatenate(
        [_pexp(W1rt[:H], 4, H), _pexp(W1rt[H:], 4, H),
         _pexp(Whh1r.T, 4, H)], axis=0)
    b1f = _pbias(bih1f + bhh1f, 4, H)
    b1r = _pbias(bih1r + bhh1r, 4, H)
    hf1, hb1 = _run_l1(hf0, hb0, W1f, b1f, W1r, b1r)

    wf2 = jnp.concatenate([w_att[:H], w_att[:H]]).reshape(1, 1, Hp)
    wr2 = jnp.concatenate([w_att[H:], w_att[H:]]).reshape(1, 1, Hp)
    Pm = jnp.zeros((Hp, Hp), F32)
    Pm = Pm.at[:H, :H].set(1.0).at[H:, H:].set(1.0)
    w1t = fc1_w.T
    w1a = _pexp(w1t[:H], 1, H2)
    w1b = _pexp(w1t[H:], 1, H2)
    b1p = _pbias(fc1_b, 1, H2)
    w2m = _pexp(fc_w.T, 1, C)
    b2p = _pbias(fc_b, 1, C)
    out2 = _run_att(hf1, hb1, wf2, wr2, Pm, w1a, w1b, b1p, w2m, b2p)
    return out2.reshape(B, C)


def kernel(x, emb, Wih0f, Whh0f, bih0f, bhh0f, Wih0r, Whh0r, bih0r, bhh0r,
           Wih1f, Whh1f, bih1f, bhh1f, Wih1r, Whh1r, bih1r, bhh1r,
           w_att, fc1_w, fc1_b, fc_w, fc_b):
    idx = x.astype(jnp.int32).T.reshape(-1)
    e2 = _make_sc_gather()(emb, idx).reshape(L, Bp, Hp)
    return _tc_forward(e2, Wih0f, Whh0f, bih0f, bhh0f, Wih0r, Whh0r,
                       bih0r, bhh0r, Wih1f, Whh1f, bih1f, bhh1f,
                       Wih1r, Whh1r, bih1r, bhh1r,
                       w_att, fc1_w, fc1_b, fc_w, fc_b)
